# baseline (speedup 1.0000x reference)
import functools

import jax
import jax.numpy as jnp
from jax import lax
from jax.experimental import pallas as pl
from jax.experimental.pallas import tpu as pltpu
from jax.experimental.pallas import tpu_sc as plsc

DIM = 32


def _make_gather(BATCH: int, HIST: int):
    info = plsc.get_sparse_core_info()
    NC, NS, L = info.num_cores, info.num_subcores, info.num_lanes
    NW = NC * NS
    assert BATCH % (NW * L) == 0 and HIST % 2 == 0
    bw = BATCH // NW
    toks_w = bw * HIST
    nblk = bw // L

    mesh = plsc.VectorSubcoreMesh(core_axis_name="c", subcore_axis_name="s")

    @functools.partial(
        pl.kernel,
        out_type=jax.ShapeDtypeStruct((HIST, DIM, BATCH), jnp.float32),
        mesh=mesh,
        scratch_types=[
            pltpu.VMEM((toks_w,), jnp.int32),
            pltpu.VMEM((HIST, bw), jnp.int32),
            pltpu.VMEM((bw, DIM), jnp.float32),
            pltpu.VMEM((bw, DIM), jnp.float32),
            pltpu.VMEM((DIM, bw), jnp.float32),
            pltpu.SemaphoreType.DMA,
            pltpu.SemaphoreType.DMA,
        ],
        compiler_params=pltpu.CompilerParams(
            use_tc_tiling_on_sc=False, needs_layout_passes=False),
    )
    def emb(idx_hbm, table_hbm, out_hbm, idx_v, tok_t, g0, g1, ob,
            sem0, sem1):
        wid = lax.axis_index("s") * NC + lax.axis_index("c")
        pltpu.sync_copy(idx_hbm.at[pl.ds(wid * toks_w, toks_w)], idx_v)

        iota = lax.iota(jnp.int32, 16)
        iota_h = iota * HIST

        def transform(h, carry):
            for blk in range(nblk):
                src = iota_h + (blk * 16 * HIST) + h
                tok_t[h, pl.ds(blk * 16, 16)] = plsc.load_gather(idx_v, [src])
            return carry

        lax.fori_loop(0, HIST, transform, 0)

        def fire(h, buf, sem):
            pltpu.async_copy(table_hbm.at[tok_t.at[h]], buf, sem)

        def drain(buf, sem):
            pltpu.make_async_copy(table_hbm.at[tok_t.at[0]], buf, sem).wait()

        def extract_write(h, buf):
            for blk in range(nblk):
                rows = iota + blk * 16
                for d in range(DIM):
                    ob[d, pl.ds(blk * 16, 16)] = plsc.load_gather(
                        buf, [rows, jnp.full((16,), d, jnp.int32)])
            pltpu.sync_copy(ob, out_hbm.at[h, :, pl.ds(wid * bw, bw)])

        fire(0, g0, sem0)

        def body(g, carry):
            fire(2 * g + 1, g1, sem1)
            drain(g0, sem0)
            extract_write(2 * g, g0)

            @pl.when(2 * g + 2 < HIST)
            def _():
                fire(2 * g + 2, g0, sem0)

            drain(g1, sem1)
            extract_write(2 * g + 1, g1)
            return carry

        lax.fori_loop(0, HIST // 2, body, 0)

    return emb


def kernel(token_ids, embed):
    BATCH, HIST = token_ids.shape
    idx = token_ids.reshape(-1).astype(jnp.int32)
    out = _make_gather(BATCH, HIST)(idx, embed)
    return jnp.transpose(out, (2, 0, 1))


# speedup vs baseline: 1.3157x; 1.3157x over previous
import functools

import jax
import jax.numpy as jnp
from jax import lax
from jax.experimental import pallas as pl
from jax.experimental.pallas import tpu as pltpu
from jax.experimental.pallas import tpu_sc as plsc

DIM = 32


def _make_gather(BATCH: int, HIST: int):
    info = plsc.get_sparse_core_info()
    NC, NS, L = info.num_cores, info.num_subcores, info.num_lanes
    NW = NC * NS
    assert BATCH % (NW * L) == 0 and HIST % 2 == 0
    bw = BATCH // NW
    toks_w = bw * HIST
    nblk = bw // L

    mesh = plsc.VectorSubcoreMesh(core_axis_name="c", subcore_axis_name="s")

    @functools.partial(
        pl.kernel,
        out_type=jax.ShapeDtypeStruct((HIST, DIM, BATCH), jnp.float32),
        mesh=mesh,
        scratch_types=[
            pltpu.VMEM((toks_w,), jnp.int32),
            pltpu.VMEM((HIST, bw), jnp.int32),
            pltpu.VMEM((bw, DIM), jnp.float32),
            pltpu.VMEM((bw, DIM), jnp.float32),
            pltpu.VMEM((DIM, bw), jnp.float32),
            pltpu.SemaphoreType.DMA,
            pltpu.SemaphoreType.DMA,
        ],
        compiler_params=pltpu.CompilerParams(
            use_tc_tiling_on_sc=False, needs_layout_passes=False),
    )
    def emb(idx_hbm, table_hbm, out_hbm, idx_v, tok_t, g0, g1, ob,
            sem0, sem1):
        wid = lax.axis_index("s") * NC + lax.axis_index("c")
        pltpu.sync_copy(idx_hbm.at[pl.ds(wid * toks_w, toks_w)], idx_v)

        iota = lax.iota(jnp.int32, 16)
        iota_h = iota * HIST

        def transform(h, carry):
            for blk in range(nblk):
                src = iota_h + (blk * 16 * HIST) + h
                tok_t[h, pl.ds(blk * 16, 16)] = plsc.load_gather(idx_v, [src])
            return carry

        lax.fori_loop(0, HIST, transform, 0)

        def fire(h, buf, sem):
            pltpu.async_copy(table_hbm.at[tok_t.at[h]], buf, sem)

        def drain(buf, sem):
            pltpu.make_async_copy(table_hbm.at[tok_t.at[0]], buf, sem).wait()

        def extract_write(h, buf):
            for blk in range(nblk):
                rows = iota + blk * 16
                vals = [
                    plsc.load_gather(buf, [rows, jnp.full((16,), d, jnp.int32)])
                    for d in range(DIM)
                ]
                for d in range(DIM):
                    ob[d, pl.ds(blk * 16, 16)] = vals[d]
            pltpu.sync_copy(ob, out_hbm.at[h, :, pl.ds(wid * bw, bw)])

        fire(0, g0, sem0)

        def body(g, carry):
            fire(2 * g + 1, g1, sem1)
            drain(g0, sem0)
            extract_write(2 * g, g0)

            @pl.when(2 * g + 2 < HIST)
            def _():
                fire(2 * g + 2, g0, sem0)

            drain(g1, sem1)
            extract_write(2 * g + 1, g1)
            return carry

        lax.fori_loop(0, HIST // 2, body, 0)

    return emb


def kernel(token_ids, embed):
    BATCH, HIST = token_ids.shape
    idx = token_ids.reshape(-1).astype(jnp.int32)
    out = _make_gather(BATCH, HIST)(idx, embed)
    return jnp.transpose(out, (2, 0, 1))


# speedup vs baseline: 1.3758x; 1.0457x over previous
import functools

import jax
import jax.numpy as jnp
from jax import lax
from jax.experimental import pallas as pl
from jax.experimental.pallas import tpu as pltpu
from jax.experimental.pallas import tpu_sc as plsc

DIM = 32
NB = 4


def _make_gather(BATCH: int, HIST: int):
    info = plsc.get_sparse_core_info()
    NC, NS, L = info.num_cores, info.num_subcores, info.num_lanes
    NW = NC * NS
    assert BATCH % (NW * L) == 0 and HIST % NB == 0
    bw = BATCH // NW
    toks_w = bw * HIST
    nblk = bw // L
    n_grp = HIST // NB

    mesh = plsc.VectorSubcoreMesh(core_axis_name="c", subcore_axis_name="s")

    @functools.partial(
        pl.kernel,
        out_type=jax.ShapeDtypeStruct((HIST, DIM, BATCH), jnp.float32),
        mesh=mesh,
        scratch_types=(
            [pltpu.VMEM((toks_w,), jnp.int32),
             pltpu.VMEM((HIST, bw), jnp.int32)]
            + [pltpu.VMEM((bw, DIM), jnp.float32) for _ in range(NB)]
            + [pltpu.VMEM((DIM, bw), jnp.float32) for _ in range(NB)]
            + [pltpu.SemaphoreType.DMA for _ in range(2 * NB)]
        ),
        compiler_params=pltpu.CompilerParams(
            use_tc_tiling_on_sc=False, needs_layout_passes=False),
    )
    def emb(idx_hbm, table_hbm, out_hbm, idx_v, tok_t, *rest):
        g = rest[:NB]
        ob = rest[NB:2 * NB]
        gsem = rest[2 * NB:3 * NB]
        wsem = rest[3 * NB:4 * NB]
        wid = lax.axis_index("s") * NC + lax.axis_index("c")
        pltpu.sync_copy(idx_hbm.at[pl.ds(wid * toks_w, toks_w)], idx_v)

        iota = lax.iota(jnp.int32, 16)
        iota_h = iota * HIST

        def transform(h, carry):
            vals = [
                plsc.load_gather(idx_v, [iota_h + (blk * 16 * HIST) + h])
                for blk in range(nblk)
            ]
            for blk in range(nblk):
                tok_t[h, pl.ds(blk * 16, 16)] = vals[blk]
            return carry

        lax.fori_loop(0, HIST, transform, 0)

        def fire(h, b):
            pltpu.async_copy(table_hbm.at[tok_t.at[h]], g[b], gsem[b])

        def gdrain(b):
            pltpu.make_async_copy(table_hbm.at[tok_t.at[0]], g[b],
                                  gsem[b]).wait()

        def wfire(h, b):
            pltpu.async_copy(ob[b], out_hbm.at[h, :, pl.ds(wid * bw, bw)],
                             wsem[b])

        def wdrain(b):
            pltpu.make_async_copy(ob[b],
                                  out_hbm.at[0, :, pl.ds(wid * bw, bw)],
                                  wsem[b]).wait()

        def extract(buf, b):
            for blk in range(nblk):
                rows = iota + blk * 16
                vals = [
                    plsc.load_gather(
                        buf, [rows, jnp.full((16,), d, jnp.int32)])
                    for d in range(DIM)
                ]
                for d in range(DIM):
                    ob[b][d, pl.ds(blk * 16, 16)] = vals[d]

        for b in range(NB):
            fire(b, b)

        def body(grp, carry):
            h0 = grp * NB
            for b in range(NB):
                @pl.when(grp > 0)
                def _():
                    wdrain(b)

                gdrain(b)
                extract(g[b], b)
                wfire(h0 + b, b)

                @pl.when(h0 + b + NB < HIST)
                def _():
                    fire(h0 + b + NB, b)
            return carry

        lax.fori_loop(0, n_grp, body, 0)
        for b in range(NB):
            wdrain(b)

    return emb


def kernel(token_ids, embed):
    BATCH, HIST = token_ids.shape
    idx = token_ids.reshape(-1).astype(jnp.int32)
    out = _make_gather(BATCH, HIST)(idx, embed)
    return jnp.transpose(out, (2, 0, 1))
